# Initial kernel scaffold; baseline (speedup 1.0000x reference)
import jax
import jax.numpy as jnp
from jax.experimental import pallas as pl


def kernel(x, edge_index, edge_attr, fc1_w, fc1_b, g_all, mu_all, sigma_all, root_all, cbias_all, fc2_w, fc2_b, fc3_w, fc3_b):
    raise NotImplementedError("write your pallas kernel here")


# speedup vs baseline: 5.0719x; 5.0719x over previous
import functools

import jax
import jax.numpy as jnp
from jax import lax
from jax.experimental import pallas as pl
from jax.experimental.pallas import tpu as pltpu
from jax.experimental.pallas import tpu_sc as plsc

N = 10000
E = 320000
IN_DIM = 128
OUT_DIM = 1
ED = 4
W = 32
K = 3
DEPTH = 4
NCONV = 8
EPS = 1e-15

LANES = 128
ROWS = E // LANES
NT = 32
RPT = ROWS // NT
TAIL = ROWS - NT * RPT
NPT = N // 16
RBLK = 2000
GRID = N // RBLK


def _gau_body(ea_ref, mu_ref, coef_ref, out_ref):
    for k in range(K):
        acc = None
        for d in range(ED):
            t = ea_ref[d] - mu_ref[0, k, d]
            term = coef_ref[0, k, d] * t * t
            acc = term if acc is None else acc + term
        out_ref[0, k] = jnp.exp(acc)


def _gau_table(ea_t, mu_all, coef_all):
    return pl.pallas_call(
        _gau_body,
        grid=(NCONV,),
        in_specs=[
            pl.BlockSpec((ED, ROWS, LANES), lambda j: (0, 0, 0)),
            pl.BlockSpec((1, K, ED), lambda j: (j, 0, 0), memory_space=pltpu.SMEM),
            pl.BlockSpec((1, K, ED), lambda j: (j, 0, 0), memory_space=pltpu.SMEM),
        ],
        out_specs=pl.BlockSpec((1, K, ROWS, LANES), lambda j: (j, 0, 0, 0)),
        out_shape=jax.ShapeDtypeStruct((NCONV, K, ROWS, LANES), jnp.float32),
    )(ea_t, mu_all, coef_all)


def _prep_body(x_ref, w1_ref, b1_ref, g_ref, h_ref, hg_ref):
    h = jnp.dot(x_ref[...], w1_ref[...], preferred_element_type=jnp.float32)
    h = h + b1_ref[0]
    h_ref[...] = h
    hg_ref[...] = jnp.dot(h, g_ref[...], preferred_element_type=jnp.float32)


def _prep(x, w1, b1, g0):
    return pl.pallas_call(
        _prep_body,
        grid=(GRID,),
        in_specs=[
            pl.BlockSpec((RBLK, IN_DIM), lambda i: (i, 0)),
            pl.BlockSpec((IN_DIM, W), lambda i: (0, 0)),
            pl.BlockSpec((1, W), lambda i: (0, 0)),
            pl.BlockSpec((W, K * W), lambda i: (0, 0)),
        ],
        out_specs=[
            pl.BlockSpec((RBLK, W), lambda i: (i, 0)),
            pl.BlockSpec((RBLK, K * W), lambda i: (i, 0)),
        ],
        out_shape=[
            jax.ShapeDtypeStruct((N, W), jnp.float32),
            jax.ShapeDtypeStruct((N, K * W), jnp.float32),
        ],
    )(x, w1, b1, g0)


def _mid_body(with_res, agg_ref, deg_ref, h_ref, res_ref, root_ref, cb_ref,
              gn_ref, ho_ref, hg_ref):
    agg = agg_ref[0] + agg_ref[1]
    deg = jnp.maximum(deg_ref[0, :, 0:1] + deg_ref[1, :, 0:1], 1.0)
    t = agg / deg
    t = t + jnp.dot(h_ref[...], root_ref[...], preferred_element_type=jnp.float32)
    t = t + cb_ref[0]
    if with_res:
        t = t + res_ref[...]
    h_out = jax.nn.gelu(t)
    ho_ref[...] = h_out
    hg_ref[...] = jnp.dot(h_out, gn_ref[...], preferred_element_type=jnp.float32)


def _mid(with_res, wn, agg2, deg8, h_in, res, root, cb, g_next):
    return pl.pallas_call(
        functools.partial(_mid_body, with_res),
        grid=(GRID,),
        in_specs=[
            pl.BlockSpec((2, RBLK, W), lambda i: (0, i, 0)),
            pl.BlockSpec((2, RBLK, 8), lambda i: (0, i, 0)),
            pl.BlockSpec((RBLK, W), lambda i: (i, 0)),
            pl.BlockSpec((RBLK, W), lambda i: (i, 0)),
            pl.BlockSpec((W, W), lambda i: (0, 0)),
            pl.BlockSpec((1, W), lambda i: (0, 0)),
            pl.BlockSpec((W, wn), lambda i: (0, 0)),
        ],
        out_specs=[
            pl.BlockSpec((RBLK, W), lambda i: (i, 0)),
            pl.BlockSpec((RBLK, wn), lambda i: (i, 0)),
        ],
        out_shape=[
            jax.ShapeDtypeStruct((N, W), jnp.float32),
            jax.ShapeDtypeStruct((N, wn), jnp.float32),
        ],
    )(agg2, deg8, h_in, res, root, cb, g_next)


def _head_body(hg_ref, b2_ref, w3_ref, b3_ref, out_ref):
    t = jax.nn.gelu(hg_ref[...] + b2_ref[0])
    out_ref[...] = jnp.dot(t, w3_ref[...], preferred_element_type=jnp.float32) + b3_ref[0]


def _head(hg128, b2, w3, b3):
    return pl.pallas_call(
        _head_body,
        grid=(GRID,),
        in_specs=[
            pl.BlockSpec((RBLK, 128), lambda i: (i, 0)),
            pl.BlockSpec((1, 128), lambda i: (0, 0)),
            pl.BlockSpec((128, OUT_DIM), lambda i: (0, 0)),
            pl.BlockSpec((1, OUT_DIM), lambda i: (0, 0)),
        ],
        out_specs=pl.BlockSpec((RBLK, OUT_DIM), lambda i: (i, 0)),
        out_shape=jax.ShapeDtypeStruct((N, OUT_DIM), jnp.float32),
    )(hg128, b2, w3, b3)


_MESH = dict(core_axis_name="c", subcore_axis_name="s")


def _conv_tec(hg_hbm, src_hbm, dst_hbm, gau_hbm, z_hbm, out_hbm,
              agg_sh, srcv, dstv, gauv, rowsv, msgv, srcT, dstT, gauT, sem):
    c = lax.axis_index("c")
    s = lax.axis_index("s")
    t = c * 16 + s
    r0 = s * NPT
    pltpu.sync_copy(z_hbm.at[pl.ds(r0, NPT)], agg_sh.at[pl.ds(r0, NPT)])
    plsc.subcore_barrier()

    start = t * RPT + jnp.minimum(t, TAIL)
    pltpu.sync_copy(src_hbm.at[pl.ds(start, RPT)], srcv)
    pltpu.sync_copy(dst_hbm.at[pl.ds(start, RPT)], dstv)
    for k in range(K):
        pltpu.sync_copy(gau_hbm.at[k, pl.ds(start, RPT)], gauv.at[k])

    def compute_chunk(src_idx, dst_idx, gau_vec):
        pltpu.async_copy(hg_hbm.at[src_idx], rowsv, sem).wait()

        def gbody(g, _):
            e0 = g * 16
            gv0 = gau_vec(0, g)
            gv1 = gau_vec(1, g)
            gv2 = gau_vec(2, g)
            for l in range(16):
                e = e0 + l
                w0 = gv0[l]
                w1 = gv1[l]
                w2 = gv2[l]
                for hh in range(2):
                    v = (w0 * rowsv[e, pl.ds(hh * 16, 16)]
                         + w1 * rowsv[e, pl.ds(W + hh * 16, 16)]
                         + w2 * rowsv[e, pl.ds(2 * W + hh * 16, 16)])
                    msgv[e, pl.ds(hh * 16, 16)] = v
            return 0

        lax.fori_loop(0, LANES // 16, gbody, 0)
        pltpu.sync_copy(msgv, agg_sh.at[dst_idx], add=True)

    def chunk(i, _):
        compute_chunk(srcv.at[i], dstv.at[i],
                      lambda k, g: gauv[k, i, pl.ds(g * 16, 16)])
        return 0

    lax.fori_loop(0, RPT, chunk, 0)

    @pl.when(t < TAIL)
    def _tail():
        rr = start + RPT
        pltpu.sync_copy(src_hbm.at[rr], srcT)
        pltpu.sync_copy(dst_hbm.at[rr], dstT)
        for k in range(K):
            pltpu.sync_copy(gau_hbm.at[k, rr], gauT.at[k])
        compute_chunk(srcT, dstT, lambda k, g: gauT[k, pl.ds(g * 16, 16)])

    plsc.subcore_barrier()
    pltpu.sync_copy(agg_sh.at[pl.ds(r0, NPT)], out_hbm.at[c, pl.ds(r0, NPT)])


def _sc_conv(hg, src2d, dst2d, gau3, zeros):
    f = pl.kernel(
        _conv_tec,
        out_type=jax.ShapeDtypeStruct((2, N, W), jnp.float32),
        mesh=plsc.VectorSubcoreMesh(**_MESH),
        scratch_types=[
            pltpu.VMEM_SHARED((N, W), jnp.float32),
            pltpu.VMEM((RPT, LANES), jnp.int32),
            pltpu.VMEM((RPT, LANES), jnp.int32),
            pltpu.VMEM((K, RPT, LANES), jnp.float32),
            pltpu.VMEM((LANES, K * W), jnp.float32),
            pltpu.VMEM((LANES, W), jnp.float32),
            pltpu.VMEM((LANES,), jnp.int32),
            pltpu.VMEM((LANES,), jnp.int32),
            pltpu.VMEM((K, LANES), jnp.float32),
            pltpu.SemaphoreType.DMA,
        ],
        compiler_params=pltpu.CompilerParams(use_tc_tiling_on_sc=False),
    )
    return f(hg, src2d, dst2d, gau3, zeros)


def _deg_tec(dst_hbm, ones_hbm, z_hbm, out_hbm, deg_sh, dstv, dstT, onesv):
    c = lax.axis_index("c")
    s = lax.axis_index("s")
    t = c * 16 + s
    r0 = s * NPT
    pltpu.sync_copy(z_hbm.at[pl.ds(r0, NPT)], deg_sh.at[pl.ds(r0, NPT)])
    pltpu.sync_copy(ones_hbm, onesv)
    plsc.subcore_barrier()

    start = t * RPT + jnp.minimum(t, TAIL)
    pltpu.sync_copy(dst_hbm.at[pl.ds(start, RPT)], dstv)

    def chunk(i, _):
        pltpu.sync_copy(onesv, deg_sh.at[dstv.at[i]], add=True)
        return 0

    lax.fori_loop(0, RPT, chunk, 0)

    @pl.when(t < TAIL)
    def _tail():
        pltpu.sync_copy(dst_hbm.at[start + RPT], dstT)
        pltpu.sync_copy(onesv, deg_sh.at[dstT], add=True)

    plsc.subcore_barrier()
    pltpu.sync_copy(deg_sh.at[pl.ds(r0, NPT)], out_hbm.at[c, pl.ds(r0, NPT)])


def _sc_deg(dst2d, ones8, zeros8):
    f = pl.kernel(
        _deg_tec,
        out_type=jax.ShapeDtypeStruct((2, N, 8), jnp.float32),
        mesh=plsc.VectorSubcoreMesh(**_MESH),
        scratch_types=[
            pltpu.VMEM_SHARED((N, 8), jnp.float32),
            pltpu.VMEM((RPT, LANES), jnp.int32),
            pltpu.VMEM((LANES,), jnp.int32),
            pltpu.VMEM((LANES, 8), jnp.float32),
        ],
        compiler_params=pltpu.CompilerParams(use_tc_tiling_on_sc=False),
    )
    return f(dst2d, ones8, zeros8)


def kernel(x, edge_index, edge_attr, fc1_w, fc1_b, g_all, mu_all, sigma_all,
           root_all, cbias_all, fc2_w, fc2_b, fc3_w, fc3_b):
    src2d = edge_index[0].reshape(ROWS, LANES)
    dst2d = edge_index[1].reshape(ROWS, LANES)
    ea_t = edge_attr.T.reshape(ED, ROWS, LANES)
    coef_all = -0.5 / (EPS + sigma_all ** 2)

    gau = _gau_table(ea_t, mu_all, coef_all)

    zeros = jnp.zeros((N, W), jnp.float32)
    zeros8 = jnp.zeros((N, 8), jnp.float32)
    ones8 = jnp.ones((LANES, 8), jnp.float32)
    deg8 = _sc_deg(dst2d, ones8, zeros8)

    b1 = fc1_b.reshape(1, W)
    b2 = fc2_b.reshape(1, 128)
    b3 = fc3_b.reshape(1, OUT_DIM)

    h, hg = _prep(x, fc1_w, b1, g_all[0])
    res = h
    for i in range(DEPTH):
        for phase in range(2):
            j = 2 * i + phase
            agg2 = _sc_conv(hg, src2d, dst2d, gau[j], zeros)
            g_next = g_all[j + 1] if j < NCONV - 1 else fc2_w
            wn = K * W if j < NCONV - 1 else 128
            cb = cbias_all[j].reshape(1, W)
            h_new, hg = _mid(phase == 1, wn, agg2, deg8, h, res,
                             root_all[j], cb, g_next)
            h = h_new
        res = h
    return _head(hg, b2, fc3_w, b3)
